# baseline (speedup 1.0000x reference)
import jax
import jax.numpy as jnp
from jax import lax
from jax.experimental import pallas as pl
from jax.experimental.pallas import tpu as pltpu

N = 4096
IN_FEAT = 512
HID_FEAT = 256
OUT_FEAT = 64
CT = 20

RE = 512
RA = 256
RD = 256

NE = N // RE
NA = N // RA
ND = N // RD

S_EF = NE
S_QS = 2 * NE
S_QF = S_QS + NA
S_ZS = S_QF + NA
S_ZF = S_ZS + NA
S_DS = S_ZF + NA
S_DF = S_DS + ND
S_END = S_DF + ND


def _mega_kernel(fs_ref, ff_ref, as_ref, af_ref, w1_ref, w2_ref,
                 wp_ref, bp_ref, wr_ref, br_ref,
                 zs_ref, zf_ref, arecs_ref, preds_ref, recs_ref,
                 arecf_ref, predf_ref, recf_ref,
                 gs_scr, gf_scr, qs_scr, qf_scr, zs_scr, zf_scr):
    i = pl.program_id(0)

    @pl.when(i < S_EF)
    def _embed_s():
        h = jnp.dot(fs_ref[...], w1_ref[...], preferred_element_type=jnp.float32)
        gs_scr[pl.ds(i * RE, RE), :] = jnp.dot(
            h, w2_ref[...], preferred_element_type=jnp.float32)

    @pl.when((i >= S_EF) & (i < S_QS))
    def _embed_f():
        h = jnp.dot(ff_ref[...], w1_ref[...], preferred_element_type=jnp.float32)
        gf_scr[pl.ds((i - S_EF) * RE, RE), :] = jnp.dot(
            h, w2_ref[...], preferred_element_type=jnp.float32)

    @pl.when((i >= S_QS) & (i < S_QF))
    def _q_s():
        qs_scr[pl.ds((i - S_QS) * RA, RA), :] = jnp.dot(
            as_ref[...], gs_scr[...], preferred_element_type=jnp.float32)

    @pl.when((i >= S_QF) & (i < S_ZS))
    def _q_f():
        qf_scr[pl.ds((i - S_QF) * RA, RA), :] = jnp.dot(
            af_ref[...], gf_scr[...], preferred_element_type=jnp.float32)

    @pl.when((i >= S_ZS) & (i < S_ZF))
    def _z_s():
        w = jnp.dot(as_ref[...], qs_scr[...], preferred_element_type=jnp.float32)
        zs_ref[...] = w
        zs_scr[pl.ds((i - S_ZS) * RA, RA), :] = w

    @pl.when((i >= S_ZF) & (i < S_DS))
    def _z_f():
        w = jnp.dot(af_ref[...], qf_scr[...], preferred_element_type=jnp.float32)
        zf_ref[...] = w
        zf_scr[pl.ds((i - S_ZF) * RA, RA), :] = w

    @pl.when((i >= S_DS) & (i < S_DF))
    def _dec_s():
        zb = zs_scr[pl.ds((i - S_DS) * RD, RD), :]
        prod = lax.dot_general(zb, zs_scr[...], (((1,), (1,)), ((), ())),
                               preferred_element_type=jnp.float32)
        arecs_ref[...] = jax.nn.sigmoid(prod)
        logits = jnp.dot(zb, wp_ref[...],
                         preferred_element_type=jnp.float32) + bp_ref[...]
        preds_ref[...] = jax.nn.softmax(logits, axis=-1)
        recs_ref[...] = jnp.dot(zb, wr_ref[...],
                                preferred_element_type=jnp.float32) + br_ref[...]

    @pl.when(i >= S_DF)
    def _dec_f():
        zb = zf_scr[pl.ds((i - S_DF) * RD, RD), :]
        prod = lax.dot_general(zb, zf_scr[...], (((1,), (1,)), ((), ())),
                               preferred_element_type=jnp.float32)
        arecf_ref[...] = jax.nn.sigmoid(prod)
        logits = jnp.dot(zb, wp_ref[...],
                         preferred_element_type=jnp.float32) + bp_ref[...]
        predf_ref[...] = jax.nn.softmax(logits, axis=-1)
        recf_ref[...] = jnp.dot(zb, wr_ref[...],
                                preferred_element_type=jnp.float32) + br_ref[...]


def _fs_idx(i):
    return (jnp.minimum(i, NE - 1), 0)


def _ff_idx(i):
    return (jnp.clip(i - S_EF, 0, NE - 1), 0)


def _as_idx(i):
    blk = jnp.where(i < S_QS, 0,
          jnp.where(i < S_QF, i - S_QS,
          jnp.where(i < S_ZS, 0,
          jnp.where(i < S_ZF, i - S_ZS, NA - 1))))
    return (blk, 0)


def _af_idx(i):
    blk = jnp.where(i < S_QF, 0,
          jnp.where(i < S_ZS, i - S_QF,
          jnp.where(i < S_ZF, 0,
          jnp.where(i < S_DS, i - S_ZF, NA - 1))))
    return (blk, 0)


def _zs_idx(i):
    return (jnp.clip(i - S_ZS, 0, NA - 1), 0)


def _zf_idx(i):
    return (jnp.clip(i - S_ZF, 0, NA - 1), 0)


def _ds_idx(i):
    return (jnp.clip(i - S_DS, 0, ND - 1), 0)


def _df_idx(i):
    return (jnp.clip(i - S_DF, 0, ND - 1), 0)


def _pin(i):
    return (0, 0)


def kernel(features, features_sc, adj_spatial, adj_feature, W1, W2, Wp, bp, Wr, br):
    bp2 = bp.reshape(1, CT)
    br2 = br.reshape(1, IN_FEAT)

    f32 = jnp.float32
    outs = pl.pallas_call(
        _mega_kernel,
        grid=(S_END,),
        in_specs=[
            pl.BlockSpec((RE, IN_FEAT), _fs_idx),
            pl.BlockSpec((RE, IN_FEAT), _ff_idx),
            pl.BlockSpec((RA, N), _as_idx),
            pl.BlockSpec((RA, N), _af_idx),
            pl.BlockSpec((IN_FEAT, HID_FEAT), _pin),
            pl.BlockSpec((HID_FEAT, OUT_FEAT), _pin),
            pl.BlockSpec((OUT_FEAT, CT), _pin),
            pl.BlockSpec((1, CT), _pin),
            pl.BlockSpec((OUT_FEAT, IN_FEAT), _pin),
            pl.BlockSpec((1, IN_FEAT), _pin),
        ],
        out_specs=[
            pl.BlockSpec((RA, OUT_FEAT), _zs_idx),
            pl.BlockSpec((RA, OUT_FEAT), _zf_idx),
            pl.BlockSpec((RD, N), _ds_idx),
            pl.BlockSpec((RD, CT), _ds_idx),
            pl.BlockSpec((RD, IN_FEAT), _ds_idx),
            pl.BlockSpec((RD, N), _df_idx),
            pl.BlockSpec((RD, CT), _df_idx),
            pl.BlockSpec((RD, IN_FEAT), _df_idx),
        ],
        out_shape=[
            jax.ShapeDtypeStruct((N, OUT_FEAT), f32),
            jax.ShapeDtypeStruct((N, OUT_FEAT), f32),
            jax.ShapeDtypeStruct((N, N), f32),
            jax.ShapeDtypeStruct((N, CT), f32),
            jax.ShapeDtypeStruct((N, IN_FEAT), f32),
            jax.ShapeDtypeStruct((N, N), f32),
            jax.ShapeDtypeStruct((N, CT), f32),
            jax.ShapeDtypeStruct((N, IN_FEAT), f32),
        ],
        scratch_shapes=[
            pltpu.VMEM((N, OUT_FEAT), f32),
            pltpu.VMEM((N, OUT_FEAT), f32),
            pltpu.VMEM((N, OUT_FEAT), f32),
            pltpu.VMEM((N, OUT_FEAT), f32),
            pltpu.VMEM((N, OUT_FEAT), f32),
            pltpu.VMEM((N, OUT_FEAT), f32),
        ],
        compiler_params=pltpu.CompilerParams(
            dimension_semantics=("arbitrary",),
            vmem_limit_bytes=100 * 1024 * 1024,
        ),
    )(features, features_sc, adj_spatial, adj_feature, W1, W2, Wp, bp2, Wr, br2)

    z_s, z_f, arec_s, pred_s, rec_s, arec_f, pred_f, rec_f = outs
    return (z_s, z_f, rec_s, rec_f, arec_s, arec_f, pred_s, pred_f)


# speedup vs baseline: 1.1662x; 1.1662x over previous
import jax
import jax.numpy as jnp
from jax import lax
from jax.experimental import pallas as pl
from jax.experimental.pallas import tpu as pltpu

N = 4096
IN_FEAT = 512
HID_FEAT = 256
OUT_FEAT = 64
CT = 20

RB = 512
NB = N // RB

S_EF = NB
S_QS = 2 * NB
S_QF = 3 * NB
S_ZS = 4 * NB
S_ZF = 5 * NB
S_END = 6 * NB


def _encode_kernel(fs_ref, ff_ref, as_ref, af_ref, w1_ref, w2_ref,
                   zs_ref, zf_ref,
                   gs_scr, gf_scr, qs_scr, qf_scr):
    i = pl.program_id(0)

    @pl.when(i < S_EF)
    def _embed_s():
        h = jnp.dot(fs_ref[...], w1_ref[...], preferred_element_type=jnp.float32)
        gs_scr[pl.ds(i * RB, RB), :] = jnp.dot(
            h, w2_ref[...], preferred_element_type=jnp.float32)

    @pl.when((i >= S_EF) & (i < S_QS))
    def _embed_f():
        h = jnp.dot(ff_ref[...], w1_ref[...], preferred_element_type=jnp.float32)
        gf_scr[pl.ds((i - S_EF) * RB, RB), :] = jnp.dot(
            h, w2_ref[...], preferred_element_type=jnp.float32)

    @pl.when((i >= S_QS) & (i < S_QF))
    def _q_s():
        qs_scr[pl.ds((i - S_QS) * RB, RB), :] = jnp.dot(
            as_ref[...], gs_scr[...], preferred_element_type=jnp.float32)

    @pl.when((i >= S_QF) & (i < S_ZS))
    def _q_f():
        qf_scr[pl.ds((i - S_QF) * RB, RB), :] = jnp.dot(
            af_ref[...], gf_scr[...], preferred_element_type=jnp.float32)

    @pl.when((i >= S_ZS) & (i < S_ZF))
    def _z_s():
        zs_ref[...] = jnp.dot(as_ref[...], qs_scr[...],
                              preferred_element_type=jnp.float32)

    @pl.when(i >= S_ZF)
    def _z_f():
        zf_ref[...] = jnp.dot(af_ref[...], qf_scr[...],
                              preferred_element_type=jnp.float32)


def _fs_idx(i):
    return (jnp.minimum(i, NB - 1), 0)


def _ff_idx(i):
    return (jnp.clip(i - S_EF, 0, NB - 1), 0)


def _as_idx(i):
    blk = jnp.where(i < S_QS, 0,
          jnp.where(i < S_QF, i - S_QS,
          jnp.where(i < S_ZS, 0,
          jnp.where(i < S_ZF, i - S_ZS, NB - 1))))
    return (blk, 0)


def _af_idx(i):
    blk = jnp.where(i < S_QF, 0,
          jnp.where(i < S_ZS, i - S_QF,
          jnp.where(i < S_ZF, 0, i - S_ZF)))
    return (blk, 0)


def _zs_idx(i):
    return (jnp.clip(i - S_ZS, 0, NB - 1), 0)


def _zf_idx(i):
    return (jnp.clip(i - S_ZF, 0, NB - 1), 0)


def _pin(i):
    return (0, 0)


def _encode(feat_s, feat_f, adj_s, adj_f, W1, W2):
    f32 = jnp.float32
    return pl.pallas_call(
        _encode_kernel,
        grid=(S_END,),
        in_specs=[
            pl.BlockSpec((RB, IN_FEAT), _fs_idx),
            pl.BlockSpec((RB, IN_FEAT), _ff_idx),
            pl.BlockSpec((RB, N), _as_idx),
            pl.BlockSpec((RB, N), _af_idx),
            pl.BlockSpec((IN_FEAT, HID_FEAT), _pin),
            pl.BlockSpec((HID_FEAT, OUT_FEAT), _pin),
        ],
        out_specs=[
            pl.BlockSpec((RB, OUT_FEAT), _zs_idx),
            pl.BlockSpec((RB, OUT_FEAT), _zf_idx),
        ],
        out_shape=[
            jax.ShapeDtypeStruct((N, OUT_FEAT), f32),
            jax.ShapeDtypeStruct((N, OUT_FEAT), f32),
        ],
        scratch_shapes=[
            pltpu.VMEM((N, OUT_FEAT), f32),
            pltpu.VMEM((N, OUT_FEAT), f32),
            pltpu.VMEM((N, OUT_FEAT), f32),
            pltpu.VMEM((N, OUT_FEAT), f32),
        ],
        compiler_params=pltpu.CompilerParams(
            dimension_semantics=("arbitrary",),
            vmem_limit_bytes=100 * 1024 * 1024,
        ),
    )(feat_s, feat_f, adj_s, adj_f, W1, W2)


def _decode_one(zb, z_all, wp, bp, wr, br, arec_ref, pred_ref, rec_ref):
    prod = lax.dot_general(zb, z_all, (((1,), (1,)), ((), ())),
                           preferred_element_type=jnp.float32)
    arec_ref[...] = jax.nn.sigmoid(prod)
    logits = jnp.dot(zb, wp, preferred_element_type=jnp.float32) + bp
    pred_ref[...] = jax.nn.softmax(logits, axis=-1)
    rec_ref[...] = jnp.dot(zb, wr, preferred_element_type=jnp.float32) + br


def _decode_kernel(zbs_ref, zbf_ref, zs_ref, zf_ref, wp_ref, bp_ref, wr_ref,
                   br_ref, arecs_ref, preds_ref, recs_ref,
                   arecf_ref, predf_ref, recf_ref):
    wp = wp_ref[...]
    bp = bp_ref[...]
    wr = wr_ref[...]
    br = br_ref[...]
    _decode_one(zbs_ref[...], zs_ref[...], wp, bp, wr, br,
                arecs_ref, preds_ref, recs_ref)
    _decode_one(zbf_ref[...], zf_ref[...], wp, bp, wr, br,
                arecf_ref, predf_ref, recf_ref)


def _blk(i):
    return (i, 0)


def _decode(z_s, z_f, Wp, bp2, Wr, br2):
    f32 = jnp.float32
    return pl.pallas_call(
        _decode_kernel,
        grid=(NB,),
        in_specs=[
            pl.BlockSpec((RB, OUT_FEAT), _blk),
            pl.BlockSpec((RB, OUT_FEAT), _blk),
            pl.BlockSpec((N, OUT_FEAT), _pin),
            pl.BlockSpec((N, OUT_FEAT), _pin),
            pl.BlockSpec((OUT_FEAT, CT), _pin),
            pl.BlockSpec((1, CT), _pin),
            pl.BlockSpec((OUT_FEAT, IN_FEAT), _pin),
            pl.BlockSpec((1, IN_FEAT), _pin),
        ],
        out_specs=[
            pl.BlockSpec((RB, N), _blk),
            pl.BlockSpec((RB, CT), _blk),
            pl.BlockSpec((RB, IN_FEAT), _blk),
            pl.BlockSpec((RB, N), _blk),
            pl.BlockSpec((RB, CT), _blk),
            pl.BlockSpec((RB, IN_FEAT), _blk),
        ],
        out_shape=[
            jax.ShapeDtypeStruct((N, N), f32),
            jax.ShapeDtypeStruct((N, CT), f32),
            jax.ShapeDtypeStruct((N, IN_FEAT), f32),
            jax.ShapeDtypeStruct((N, N), f32),
            jax.ShapeDtypeStruct((N, CT), f32),
            jax.ShapeDtypeStruct((N, IN_FEAT), f32),
        ],
        compiler_params=pltpu.CompilerParams(
            dimension_semantics=("arbitrary",),
            vmem_limit_bytes=100 * 1024 * 1024,
        ),
    )(z_s, z_f, z_s, z_f, Wp, bp2, Wr, br2)


def kernel(features, features_sc, adj_spatial, adj_feature, W1, W2, Wp, bp, Wr, br):
    bp2 = bp.reshape(1, CT)
    br2 = br.reshape(1, IN_FEAT)

    z_s, z_f = _encode(features, features_sc, adj_spatial, adj_feature, W1, W2)
    arec_s, pred_s, rec_s, arec_f, pred_f, rec_f = _decode(
        z_s, z_f, Wp, bp2, Wr, br2)

    return (z_s, z_f, rec_s, rec_f, arec_s, arec_f, pred_s, pred_f)
